# baseline (speedup 1.0000x reference)
import functools
import math

import jax
import jax.numpy as jnp
from jax import lax
from jax.experimental import pallas as pl
from jax.experimental.pallas import tpu as pltpu
from jax.experimental.pallas import tpu_sc as plsc

_NW = 32
_L = 16

_EC = 1024
_AC = 512


def _expand_chunk(idx_ref, table_ref, rows_ref, nrows, ncols):
    iota = lax.broadcasted_iota(jnp.int32, (_L,), 0)
    mask = ncols - 1
    grp = 4
    transposed = len(rows_ref.shape) == 2

    def gbody(g, carry):
        ev = idx_ref[pl.ds(g * _L, _L)]
        rows = g * _L + iota
        fbase = rows * ncols
        for c0 in range(0, ncols, grp):
            colvs = [lax.bitwise_and(c0 + dc + iota, mask)
                     for dc in range(grp)]
            vals = [plsc.load_gather(table_ref, [ev, colv])
                    for colv in colvs]
            for colv, v in zip(colvs, vals):
                if transposed:
                    plsc.store_scatter(rows_ref, [colv, rows], v)
                else:
                    plsc.store_scatter(rows_ref, [fbase + colv], v)
        return carry

    lax.fori_loop(0, nrows // _L, gbody, 0)


def _sc_gathers(a, e, atom_table, edge_table):
    n_nodes = a.shape[0]
    n_edges = e.shape[0]
    av, adim = atom_table.shape
    ev_, edim = edge_table.shape
    e_chunks = ((n_edges + _EC - 1) // _EC + _NW - 1) // _NW
    a_chunks_per_w = (n_nodes + _AC * _NW - 1) // (_AC * _NW)
    mesh = plsc.VectorSubcoreMesh(core_axis_name="c", subcore_axis_name="s")

    @functools.partial(
        pl.kernel,
        out_type=(
            jax.ShapeDtypeStruct((n_nodes * adim,), jnp.float32),
            jax.ShapeDtypeStruct((edim, n_edges), jnp.float32),
        ),
        mesh=mesh,
        compiler_params=pltpu.CompilerParams(
            use_tc_tiling_on_sc=False, needs_layout_passes=False),
        scratch_types=[
            pltpu.VMEM((ev_, edim), jnp.float32),
            pltpu.VMEM((av, adim), jnp.float32),
            pltpu.VMEM((2, _EC), jnp.int32),
            pltpu.VMEM((2, edim, _EC), jnp.float32),
            pltpu.VMEM((_AC,), jnp.int32),
            pltpu.VMEM((_AC * adim,), jnp.float32),
            pltpu.SemaphoreType.DMA((2,)),
            pltpu.SemaphoreType.DMA((2,)),
            pltpu.SemaphoreType.DMA,
        ],
    )
    def k(a_hbm, e_hbm, at_hbm, et_hbm, a_out, e_out,
          et_v, at_v, eidx_v, erows_v, aidx_v, arows_v,
          sem_in, sem_out, sem_a):
        wid = lax.axis_index("s") * 2 + lax.axis_index("c")
        pltpu.sync_copy(et_hbm, et_v)
        pltpu.sync_copy(at_hbm, at_v)

        def ebase(i):
            return pl.multiple_of(
                jnp.minimum((wid + i * _NW) * _EC, n_edges - _EC), 128)

        pltpu.async_copy(e_hbm.at[pl.ds(ebase(0), _EC)], eidx_v.at[0],
                         sem_in.at[0])

        def ebody(i, carry):
            buf = lax.rem(i, 2)
            pltpu.make_async_copy(
                e_hbm.at[pl.ds(ebase(i), _EC)], eidx_v.at[buf],
                sem_in.at[buf]).wait()

            @pl.when(i + 1 < e_chunks)
            def _():
                pltpu.async_copy(
                    e_hbm.at[pl.ds(ebase(i + 1), _EC)], eidx_v.at[1 - buf],
                    sem_in.at[1 - buf])

            @pl.when(i >= 2)
            def _():
                pltpu.make_async_copy(
                    erows_v.at[buf],
                    e_out.at[:, pl.ds(ebase(i - 2), _EC)],
                    sem_out.at[buf]).wait()

            _expand_chunk(eidx_v.at[buf], et_v, erows_v.at[buf], _EC, edim)
            pltpu.async_copy(erows_v.at[buf],
                             e_out.at[:, pl.ds(ebase(i), _EC)],
                             sem_out.at[buf])
            return carry

        lax.fori_loop(0, e_chunks, ebody, 0)
        for tail in (e_chunks - 2, e_chunks - 1):
            pltpu.make_async_copy(
                erows_v.at[tail % 2],
                e_out.at[:, pl.ds(ebase(tail), _EC)],
                sem_out.at[tail % 2]).wait()

        def abody(j, carry):
            chunk = wid * a_chunks_per_w + j
            base = pl.multiple_of(
                jnp.minimum(chunk * _AC, n_nodes - _AC), 8)

            @pl.when(j > 0)
            def _():
                pltpu.make_async_copy(
                    arows_v, a_out.at[pl.ds(base * adim, _AC * adim)],
                    sem_a).wait()

            pltpu.sync_copy(a_hbm.at[pl.ds(base, _AC)], aidx_v)
            _expand_chunk(aidx_v, at_v, arows_v, _AC, adim)
            pltpu.async_copy(arows_v, a_out.at[pl.ds(base * adim, _AC * adim)],
                             sem_a)
            return base

        last_base = lax.fori_loop(0, a_chunks_per_w, abody, 0)
        pltpu.make_async_copy(
            arows_v, a_out.at[pl.ds(last_base * adim, _AC * adim)],
            sem_a).wait()

    return k(a, e, atom_table, edge_table)


def _tc_cond(t_col, batch2d, node_count_table, W_t, b_row):
    num_graphs = t_col.shape[0]
    ncv, ncd = node_count_table.shape
    tdim = W_t.shape[0]
    half = tdim // 2
    nblk, blk = batch2d.shape

    def body(t_ref, batch_ref, nct_ref, wt_ref, bt_ref, out_ref):
        gid_col = lax.broadcasted_iota(jnp.int32, (num_graphs, 1), 0)

        def cbody(i, acc):
            row = batch_ref[pl.ds(i, 1), :]
            eq = (row == gid_col).astype(jnp.float32)
            return acc + jnp.sum(eq, axis=1, keepdims=True)

        counts = lax.fori_loop(
            0, nblk, cbody, jnp.zeros((num_graphs, 1), jnp.float32))
        n_idx = jnp.clip(counts.astype(jnp.int32), 0, ncv - 1)
        onehot = (n_idx == lax.broadcasted_iota(
            jnp.int32, (1, ncv), 1)).astype(jnp.float32)
        n_embed = jnp.dot(onehot, nct_ref[...],
                          preferred_element_type=jnp.float32,
                          precision=lax.Precision.HIGHEST)

        freqs = jnp.exp(
            (-math.log(10000.0) / half)
            * lax.broadcasted_iota(jnp.int32, (1, half), 1).astype(jnp.float32))
        args = t_ref[...] * freqs
        temb = jnp.concatenate([jnp.sin(args), jnp.cos(args)], axis=-1)
        t_embed = jnp.dot(temb, wt_ref[...],
                          preferred_element_type=jnp.float32) + bt_ref[...]

        out_ref[:, :tdim] = t_embed
        out_ref[:, tdim:] = n_embed

    return pl.pallas_call(
        body,
        out_shape=jax.ShapeDtypeStruct((num_graphs, tdim + ncd), jnp.float32),
    )(t_col, batch2d, node_count_table, W_t, b_row)


def kernel(a, e, edge_index, t, batch, atom_table, edge_table,
           node_count_table, W_t, b_t):
    del edge_index
    a2, eT = _sc_gathers(a, e, atom_table, edge_table)
    a_embed = a2.reshape(a.shape[0], atom_table.shape[1])
    e_embed = eT.T
    cond = _tc_cond(
        t.reshape(-1, 1),
        batch.reshape(25, -1),
        node_count_table,
        W_t,
        b_t.reshape(1, -1),
    )
    return a_embed, e_embed, cond


# speedup vs baseline: 17.7495x; 17.7495x over previous
import functools
import math

import jax
import jax.numpy as jnp
from jax import lax
from jax.experimental import pallas as pl
from jax.experimental.pallas import tpu as pltpu
from jax.experimental.pallas import tpu_sc as plsc

_NW = 32
_L = 16

_AC = 512

_EB = 6400


def _tc_edges(e2d, edge_table):
    nblk, blk = e2d.shape
    ev_, edim = edge_table.shape
    n_edges = nblk * blk

    def body(e_ref, et_ref, out_ref):
        ev = e_ref[0]
        tbl = et_ref[...]
        acc = jnp.zeros((edim, blk), jnp.float32)
        for v in range(ev_):
            mask = ev == v
            col = tbl[v, :].reshape(edim, 1)
            acc = jnp.where(mask, col, acc)
        out_ref[...] = acc

    return pl.pallas_call(
        body,
        grid=(nblk,),
        in_specs=[
            pl.BlockSpec((1, 1, blk), lambda i: (i, 0, 0)),
            pl.BlockSpec((ev_, edim), lambda i: (0, 0)),
        ],
        out_specs=pl.BlockSpec((edim, blk), lambda i: (0, i)),
        out_shape=jax.ShapeDtypeStruct((edim, n_edges), jnp.float32),
    )(e2d.reshape(nblk, 1, blk), edge_table)


def _expand_chunk(idx_ref, table_ref, rows_ref, nrows, ncols):
    iota = lax.broadcasted_iota(jnp.int32, (_L,), 0)
    mask = ncols - 1
    grp = 4

    def gbody(g, carry):
        ev = idx_ref[pl.ds(g * _L, _L)]
        fbase = (g * _L + iota) * ncols
        for c0 in range(0, ncols, grp):
            colvs = [lax.bitwise_and(c0 + dc + iota, mask)
                     for dc in range(grp)]
            vals = [plsc.load_gather(table_ref, [ev, colv])
                    for colv in colvs]
            for colv, v in zip(colvs, vals):
                plsc.store_scatter(rows_ref, [fbase + colv], v)
        return carry

    lax.fori_loop(0, nrows // _L, gbody, 0)


def _sc_atoms(a, atom_table):
    n_nodes = a.shape[0]
    av, adim = atom_table.shape
    a_chunks_per_w = (n_nodes + _AC * _NW - 1) // (_AC * _NW)
    mesh = plsc.VectorSubcoreMesh(core_axis_name="c", subcore_axis_name="s")

    @functools.partial(
        pl.kernel,
        out_type=jax.ShapeDtypeStruct((n_nodes * adim,), jnp.float32),
        mesh=mesh,
        compiler_params=pltpu.CompilerParams(
            use_tc_tiling_on_sc=False, needs_layout_passes=False),
        scratch_types=[
            pltpu.VMEM((av, adim), jnp.float32),
            pltpu.VMEM((_AC,), jnp.int32),
            pltpu.VMEM((2, _AC * adim), jnp.float32),
            pltpu.SemaphoreType.DMA((2,)),
        ],
    )
    def k(a_hbm, at_hbm, a_out, at_v, aidx_v, arows_v, sem_a):
        wid = lax.axis_index("s") * 2 + lax.axis_index("c")
        pltpu.sync_copy(at_hbm, at_v)

        def abody(j, carry):
            buf = lax.rem(j, 2)
            chunk = wid * a_chunks_per_w + j
            base = pl.multiple_of(
                jnp.minimum(chunk * _AC, n_nodes - _AC), 8)

            @pl.when(j > 1)
            def _():
                pltpu.make_async_copy(
                    arows_v.at[buf],
                    a_out.at[pl.ds(base * adim, _AC * adim)],
                    sem_a.at[buf]).wait()

            pltpu.sync_copy(a_hbm.at[pl.ds(base, _AC)], aidx_v)
            _expand_chunk(aidx_v, at_v, arows_v.at[buf], _AC, adim)
            pltpu.async_copy(arows_v.at[buf],
                             a_out.at[pl.ds(base * adim, _AC * adim)],
                             sem_a.at[buf])
            return base

        lax.fori_loop(0, a_chunks_per_w, abody, 0)
        for j in (a_chunks_per_w - 2, a_chunks_per_w - 1):
            chunk = wid * a_chunks_per_w + j
            base = pl.multiple_of(
                jnp.minimum(chunk * _AC, n_nodes - _AC), 8)
            pltpu.make_async_copy(
                arows_v.at[j % 2],
                a_out.at[pl.ds(base * adim, _AC * adim)],
                sem_a.at[j % 2]).wait()

    return k(a, atom_table)


def _tc_cond(t_col, batch2d, node_count_table, W_t, b_row):
    num_graphs = t_col.shape[0]
    ncv, ncd = node_count_table.shape
    tdim = W_t.shape[0]
    half = tdim // 2
    nblk, blk = batch2d.shape

    def body(t_ref, batch_ref, nct_ref, wt_ref, bt_ref, out_ref):
        gid_col = lax.broadcasted_iota(jnp.int32, (num_graphs, 1), 0)

        def cbody(i, acc):
            row = batch_ref[pl.ds(i, 1), :]
            eq = (row == gid_col).astype(jnp.float32)
            return acc + jnp.sum(eq, axis=1, keepdims=True)

        counts = lax.fori_loop(
            0, nblk, cbody, jnp.zeros((num_graphs, 1), jnp.float32))
        n_idx = jnp.clip(counts.astype(jnp.int32), 0, ncv - 1)
        onehot = (n_idx == lax.broadcasted_iota(
            jnp.int32, (1, ncv), 1)).astype(jnp.float32)
        n_embed = jnp.dot(onehot, nct_ref[...],
                          preferred_element_type=jnp.float32,
                          precision=lax.Precision.HIGHEST)

        freqs = jnp.exp(
            (-math.log(10000.0) / half)
            * lax.broadcasted_iota(jnp.int32, (1, half), 1).astype(jnp.float32))
        args = t_ref[...] * freqs
        temb = jnp.concatenate([jnp.sin(args), jnp.cos(args)], axis=-1)
        t_embed = jnp.dot(temb, wt_ref[...],
                          preferred_element_type=jnp.float32) + bt_ref[...]

        out_ref[:, :tdim] = t_embed
        out_ref[:, tdim:] = n_embed

    return pl.pallas_call(
        body,
        out_shape=jax.ShapeDtypeStruct((num_graphs, tdim + ncd), jnp.float32),
    )(t_col, batch2d, node_count_table, W_t, b_row)


def kernel(a, e, edge_index, t, batch, atom_table, edge_table,
           node_count_table, W_t, b_t):
    del edge_index
    eT = _tc_edges(e.reshape(-1, _EB), edge_table)
    e_embed = eT.T
    a2 = _sc_atoms(a, atom_table)
    a_embed = a2.reshape(a.shape[0], atom_table.shape[1])
    cond = _tc_cond(
        t.reshape(-1, 1),
        batch.reshape(25, -1),
        node_count_table,
        W_t,
        b_t.reshape(1, -1),
    )
    return a_embed, e_embed, cond


# speedup vs baseline: 19.5827x; 1.1033x over previous
import functools
import math

import jax
import jax.numpy as jnp
from jax import lax
from jax.experimental import pallas as pl
from jax.experimental.pallas import tpu as pltpu
from jax.experimental.pallas import tpu_sc as plsc

_NW = 32
_L = 16

_AC = 512

_EB = 12800


def _tc_edges(e2d, edge_table):
    nblk, blk = e2d.shape
    ev_, edim = edge_table.shape
    n_edges = nblk * blk

    def body(e_ref, et_ref, out_ref):
        ev = e_ref[0]
        onehot = (lax.broadcasted_iota(jnp.int32, (ev_, 1), 0)
                  == ev).astype(jnp.float32)
        out_ref[...] = jax.lax.dot_general(
            et_ref[...], onehot, (((0,), (0,)), ((), ())),
            preferred_element_type=jnp.float32,
            precision=lax.Precision.HIGHEST)

    return pl.pallas_call(
        body,
        grid=(nblk,),
        in_specs=[
            pl.BlockSpec((1, 1, blk), lambda i: (i, 0, 0)),
            pl.BlockSpec((ev_, edim), lambda i: (0, 0)),
        ],
        out_specs=pl.BlockSpec((edim, blk), lambda i: (0, i)),
        out_shape=jax.ShapeDtypeStruct((edim, n_edges), jnp.float32),
    )(e2d.reshape(nblk, 1, blk), edge_table)


def _expand_chunk(idx_ref, table_ref, rows_ref, nrows, ncols):
    iota = lax.broadcasted_iota(jnp.int32, (_L,), 0)
    mask = ncols - 1
    grp = 4

    def gbody(g, carry):
        ev = idx_ref[pl.ds(g * _L, _L)]
        fbase = (g * _L + iota) * ncols
        for c0 in range(0, ncols, grp):
            colvs = [lax.bitwise_and(c0 + dc + iota, mask)
                     for dc in range(grp)]
            vals = [plsc.load_gather(table_ref, [ev, colv])
                    for colv in colvs]
            for colv, v in zip(colvs, vals):
                plsc.store_scatter(rows_ref, [fbase + colv], v)
        return carry

    lax.fori_loop(0, nrows // _L, gbody, 0)


def _sc_atoms(a, atom_table):
    n_nodes = a.shape[0]
    av, adim = atom_table.shape
    a_chunks_per_w = (n_nodes + _AC * _NW - 1) // (_AC * _NW)
    mesh = plsc.VectorSubcoreMesh(core_axis_name="c", subcore_axis_name="s")

    @functools.partial(
        pl.kernel,
        out_type=jax.ShapeDtypeStruct((n_nodes * adim,), jnp.float32),
        mesh=mesh,
        compiler_params=pltpu.CompilerParams(
            use_tc_tiling_on_sc=False, needs_layout_passes=False),
        scratch_types=[
            pltpu.VMEM((av, adim), jnp.float32),
            pltpu.VMEM((_AC,), jnp.int32),
            pltpu.VMEM((2, _AC * adim), jnp.float32),
            pltpu.SemaphoreType.DMA((2,)),
        ],
    )
    def k(a_hbm, at_hbm, a_out, at_v, aidx_v, arows_v, sem_a):
        wid = lax.axis_index("s") * 2 + lax.axis_index("c")
        pltpu.sync_copy(at_hbm, at_v)

        def abody(j, carry):
            buf = lax.rem(j, 2)
            chunk = wid * a_chunks_per_w + j
            base = pl.multiple_of(
                jnp.minimum(chunk * _AC, n_nodes - _AC), 8)

            @pl.when(j > 1)
            def _():
                pltpu.make_async_copy(
                    arows_v.at[buf],
                    a_out.at[pl.ds(base * adim, _AC * adim)],
                    sem_a.at[buf]).wait()

            pltpu.sync_copy(a_hbm.at[pl.ds(base, _AC)], aidx_v)
            _expand_chunk(aidx_v, at_v, arows_v.at[buf], _AC, adim)
            pltpu.async_copy(arows_v.at[buf],
                             a_out.at[pl.ds(base * adim, _AC * adim)],
                             sem_a.at[buf])
            return base

        lax.fori_loop(0, a_chunks_per_w, abody, 0)
        for j in (a_chunks_per_w - 2, a_chunks_per_w - 1):
            chunk = wid * a_chunks_per_w + j
            base = pl.multiple_of(
                jnp.minimum(chunk * _AC, n_nodes - _AC), 8)
            pltpu.make_async_copy(
                arows_v.at[j % 2],
                a_out.at[pl.ds(base * adim, _AC * adim)],
                sem_a.at[j % 2]).wait()

    return k(a, atom_table)


def _tc_cond(t_col, batch2d, node_count_table, W_t, b_row):
    num_graphs = t_col.shape[0]
    ncv, ncd = node_count_table.shape
    tdim = W_t.shape[0]
    half = tdim // 2
    nblk, blk = batch2d.shape

    def body(t_ref, batch_ref, nct_ref, wt_ref, bt_ref, out_ref):
        gid_col = lax.broadcasted_iota(jnp.int32, (num_graphs, 1), 0)

        def cbody(i, acc):
            row = batch_ref[pl.ds(i, 1), :]
            eq = (row == gid_col).astype(jnp.float32)
            return acc + jnp.sum(eq, axis=1, keepdims=True)

        counts = lax.fori_loop(
            0, nblk, cbody, jnp.zeros((num_graphs, 1), jnp.float32))
        n_idx = jnp.clip(counts.astype(jnp.int32), 0, ncv - 1)
        onehot = (n_idx == lax.broadcasted_iota(
            jnp.int32, (1, ncv), 1)).astype(jnp.float32)
        n_embed = jnp.dot(onehot, nct_ref[...],
                          preferred_element_type=jnp.float32,
                          precision=lax.Precision.HIGHEST)

        freqs = jnp.exp(
            (-math.log(10000.0) / half)
            * lax.broadcasted_iota(jnp.int32, (1, half), 1).astype(jnp.float32))
        args = t_ref[...] * freqs
        temb = jnp.concatenate([jnp.sin(args), jnp.cos(args)], axis=-1)
        t_embed = jnp.dot(temb, wt_ref[...],
                          preferred_element_type=jnp.float32) + bt_ref[...]

        out_ref[:, :tdim] = t_embed
        out_ref[:, tdim:] = n_embed

    return pl.pallas_call(
        body,
        out_shape=jax.ShapeDtypeStruct((num_graphs, tdim + ncd), jnp.float32),
    )(t_col, batch2d, node_count_table, W_t, b_row)


def kernel(a, e, edge_index, t, batch, atom_table, edge_table,
           node_count_table, W_t, b_t):
    del edge_index
    eT = _tc_edges(e.reshape(-1, _EB), edge_table)
    e_embed = eT.T
    a2 = _sc_atoms(a, atom_table)
    a_embed = a2.reshape(a.shape[0], atom_table.shape[1])
    cond = _tc_cond(
        t.reshape(-1, 1),
        batch.reshape(25, -1),
        node_count_table,
        W_t,
        b_t.reshape(1, -1),
    )
    return a_embed, e_embed, cond


# speedup vs baseline: 22.2602x; 1.1367x over previous
import functools
import math

import jax
import jax.numpy as jnp
from jax import lax
from jax.experimental import pallas as pl
from jax.experimental.pallas import tpu as pltpu
from jax.experimental.pallas import tpu_sc as plsc

_NW = 32
_L = 16

_AC = 512

_EB = 25600


def _tc_edges(e1d, edge_table):
    ev_, edim = edge_table.shape
    n_edges = e1d.size
    blk = _EB
    nblk = (n_edges + blk - 1) // blk

    def body(e_ref, et_ref, out_ref):
        ev = e_ref[...].reshape(1, blk)
        onehot = (lax.broadcasted_iota(jnp.int32, (ev_, 1), 0)
                  == ev).astype(jnp.float32)
        out_ref[...] = jax.lax.dot_general(
            et_ref[...], onehot, (((0,), (0,)), ((), ())),
            preferred_element_type=jnp.float32,
            precision=lax.Precision.HIGHEST)

    return pl.pallas_call(
        body,
        grid=(nblk,),
        in_specs=[
            pl.BlockSpec((blk,), lambda i: (i,)),
            pl.BlockSpec((ev_, edim), lambda i: (0, 0)),
        ],
        out_specs=pl.BlockSpec((edim, blk), lambda i: (0, i)),
        out_shape=jax.ShapeDtypeStruct((edim, n_edges), jnp.float32),
    )(e1d, edge_table)


def _expand_chunk(idx_ref, table_ref, rows_ref, nrows, ncols):
    iota = lax.broadcasted_iota(jnp.int32, (_L,), 0)
    mask = ncols - 1
    grp = 4

    def gbody(g, carry):
        ev = idx_ref[pl.ds(g * _L, _L)]
        fbase = (g * _L + iota) * ncols
        for c0 in range(0, ncols, grp):
            colvs = [lax.bitwise_and(c0 + dc + iota, mask)
                     for dc in range(grp)]
            vals = [plsc.load_gather(table_ref, [ev, colv])
                    for colv in colvs]
            for colv, v in zip(colvs, vals):
                plsc.store_scatter(rows_ref, [fbase + colv], v)
        return carry

    lax.fori_loop(0, nrows // _L, gbody, 0)


def _sc_atoms(a, atom_table):
    n_nodes = a.shape[0]
    av, adim = atom_table.shape
    a_chunks_per_w = (n_nodes + _AC * _NW - 1) // (_AC * _NW)
    mesh = plsc.VectorSubcoreMesh(core_axis_name="c", subcore_axis_name="s")

    @functools.partial(
        pl.kernel,
        out_type=jax.ShapeDtypeStruct((n_nodes * adim,), jnp.float32),
        mesh=mesh,
        compiler_params=pltpu.CompilerParams(
            use_tc_tiling_on_sc=False, needs_layout_passes=False),
        scratch_types=[
            pltpu.VMEM((av, adim), jnp.float32),
            pltpu.VMEM((_AC,), jnp.int32),
            pltpu.VMEM((2, _AC * adim), jnp.float32),
            pltpu.SemaphoreType.DMA((2,)),
        ],
    )
    def k(a_hbm, at_hbm, a_out, at_v, aidx_v, arows_v, sem_a):
        wid = lax.axis_index("s") * 2 + lax.axis_index("c")
        pltpu.sync_copy(at_hbm, at_v)

        def abody(j, carry):
            buf = lax.rem(j, 2)
            chunk = wid * a_chunks_per_w + j
            base = pl.multiple_of(
                jnp.minimum(chunk * _AC, n_nodes - _AC), 8)

            @pl.when(j > 1)
            def _():
                pltpu.make_async_copy(
                    arows_v.at[buf],
                    a_out.at[pl.ds(base * adim, _AC * adim)],
                    sem_a.at[buf]).wait()

            pltpu.sync_copy(a_hbm.at[pl.ds(base, _AC)], aidx_v)
            _expand_chunk(aidx_v, at_v, arows_v.at[buf], _AC, adim)
            pltpu.async_copy(arows_v.at[buf],
                             a_out.at[pl.ds(base * adim, _AC * adim)],
                             sem_a.at[buf])
            return base

        lax.fori_loop(0, a_chunks_per_w, abody, 0)
        for j in (a_chunks_per_w - 2, a_chunks_per_w - 1):
            chunk = wid * a_chunks_per_w + j
            base = pl.multiple_of(
                jnp.minimum(chunk * _AC, n_nodes - _AC), 8)
            pltpu.make_async_copy(
                arows_v.at[j % 2],
                a_out.at[pl.ds(base * adim, _AC * adim)],
                sem_a.at[j % 2]).wait()

    return k(a, atom_table)


def _tc_cond(t_col, batch2d, node_count_table, W_t, b_row):
    num_graphs = t_col.shape[0]
    ncv, ncd = node_count_table.shape
    tdim = W_t.shape[0]
    half = tdim // 2
    nblk, blk = batch2d.shape

    def body(t_ref, batch_ref, nct_ref, wt_ref, bt_ref, out_ref):
        gid_col = lax.broadcasted_iota(jnp.int32, (num_graphs, 1), 0)

        def cbody(i, acc):
            row = batch_ref[pl.ds(i, 1), :]
            eq = (row == gid_col).astype(jnp.float32)
            return acc + jnp.sum(eq, axis=1, keepdims=True)

        counts = lax.fori_loop(
            0, nblk, cbody, jnp.zeros((num_graphs, 1), jnp.float32))
        n_idx = jnp.clip(counts.astype(jnp.int32), 0, ncv - 1)
        onehot = (n_idx == lax.broadcasted_iota(
            jnp.int32, (1, ncv), 1)).astype(jnp.float32)
        n_embed = jnp.dot(onehot, nct_ref[...],
                          preferred_element_type=jnp.float32,
                          precision=lax.Precision.HIGHEST)

        freqs = jnp.exp(
            (-math.log(10000.0) / half)
            * lax.broadcasted_iota(jnp.int32, (1, half), 1).astype(jnp.float32))
        args = t_ref[...] * freqs
        temb = jnp.concatenate([jnp.sin(args), jnp.cos(args)], axis=-1)
        t_embed = jnp.dot(temb, wt_ref[...],
                          preferred_element_type=jnp.float32) + bt_ref[...]

        out_ref[:, :tdim] = t_embed
        out_ref[:, tdim:] = n_embed

    return pl.pallas_call(
        body,
        out_shape=jax.ShapeDtypeStruct((num_graphs, tdim + ncd), jnp.float32),
    )(t_col, batch2d, node_count_table, W_t, b_row)


def kernel(a, e, edge_index, t, batch, atom_table, edge_table,
           node_count_table, W_t, b_t):
    del edge_index
    eT = _tc_edges(e, edge_table)
    e_embed = eT.T
    a2 = _sc_atoms(a, atom_table)
    a_embed = a2.reshape(a.shape[0], atom_table.shape[1])
    cond = _tc_cond(
        t.reshape(-1, 1),
        batch.reshape(25, -1),
        node_count_table,
        W_t,
        b_t.reshape(1, -1),
    )
    return a_embed, e_embed, cond
